# Initial kernel scaffold; baseline (speedup 1.0000x reference)
import jax
import jax.numpy as jnp
from jax.experimental import pallas as pl


def kernel(z1, z2, temp, edge_index):
    raise NotImplementedError("write your pallas kernel here")


# speedup vs baseline: 3.9509x; 3.9509x over previous
import functools

import jax
import jax.numpy as jnp
from jax import lax
from jax.experimental import pallas as pl
from jax.experimental.pallas import tpu as pltpu
from jax.experimental.pallas import tpu_sc as plsc

N_NODES = 10000
D_FEAT = 256
L = 16
NW = 32
CHUNK = 64
K_CHUNKS = 80
E_PAD = NW * K_CHUNKS * CHUNK
NB = CHUNK // L


def _edge_body(z1_hbm, z2_hbm, tau_hbm, e0_hbm, e1_hbm, g0_hbm, g1_hbm,
               out_hbm, z2_v, tau_v, e0_v, e1_v, g0_v, g1_v, out_v,
               r0_v, r1_v, col_v, sem0, sem1):
    wid = lax.axis_index("s") * 2 + lax.axis_index("c")
    base = wid * K_CHUNKS

    pltpu.sync_copy(z2_hbm, z2_v)
    pltpu.sync_copy(tau_hbm, tau_v)
    pltpu.sync_copy(e0_hbm.at[pl.ds(base, K_CHUNKS)], e0_v)
    pltpu.sync_copy(e1_hbm.at[pl.ds(base, K_CHUNKS)], e1_v)
    pltpu.sync_copy(g0_hbm.at[pl.ds(base, K_CHUNKS)], g0_v)
    pltpu.sync_copy(g1_hbm.at[pl.ds(base, K_CHUNKS)], g1_v)
    tau = tau_v[...]
    lane = lax.iota(jnp.int32, L)

    def chunk_body(c, carry):
        pltpu.async_copy(z1_hbm.at[e0_v.at[c]], r0_v, sem0).wait()
        pltpu.async_copy(z1_hbm.at[e1_v.at[c]], r1_v, sem1).wait()
        for b in range(NB):
            def edge_body(i, _):
                e = b * L + i
                acc = r0_v[e, pl.ds(0, L)] * r1_v[e, pl.ds(0, L)]
                for j in range(1, D_FEAT // L):
                    acc = acc + (r0_v[e, pl.ds(j * L, L)] *
                                 r1_v[e, pl.ds(j * L, L)])
                plsc.store_scatter(col_v, [lane * L + i], acc)
                return 0

            lax.fori_loop(0, L, edge_body, 0, unroll=False)
            dot = col_v[pl.ds(0, L)]
            for d2 in range(1, L):
                dot = dot + col_v[pl.ds(d2 * L, L)]
            e0b = e0_v[c, pl.ds(b * L, L)]
            e1b = e1_v[c, pl.ds(b * L, L)]
            vn = (plsc.load_gather(z2_v, [e0b]) +
                  plsc.load_gather(z2_v, [e1b]))
            g0b = g0_v[c, pl.ds(b * L, L)]
            g1b = g1_v[c, pl.ds(b * L, L)]
            flag = (dot + g0b) / tau >= g1b / tau
            sig_f = 1.0 / (1.0 + jnp.exp(-dot))
            sig_n = 1.0 / (1.0 + jnp.exp(-vn))
            out_v[c, pl.ds(b * L, L)] = jnp.where(flag, sig_f, sig_n)
        return 0

    lax.fori_loop(0, K_CHUNKS, chunk_body, 0, unroll=False)
    pltpu.sync_copy(out_v, out_hbm.at[pl.ds(base, K_CHUNKS)])


@functools.partial(jax.jit, static_argnames=())
def _decode(z1, z2f, tau16, e0, e1, g0, g1):
    mesh = plsc.VectorSubcoreMesh(core_axis_name="c", subcore_axis_name="s")
    rows2 = E_PAD // CHUNK
    grid_kernel = pl.kernel(
        _edge_body,
        out_type=jax.ShapeDtypeStruct((rows2, CHUNK), jnp.float32),
        mesh=mesh,
        scratch_types=[
            pltpu.VMEM((N_NODES,), jnp.float32),
            pltpu.VMEM((L,), jnp.float32),
            pltpu.VMEM((K_CHUNKS, CHUNK), jnp.int32),
            pltpu.VMEM((K_CHUNKS, CHUNK), jnp.int32),
            pltpu.VMEM((K_CHUNKS, CHUNK), jnp.float32),
            pltpu.VMEM((K_CHUNKS, CHUNK), jnp.float32),
            pltpu.VMEM((K_CHUNKS, CHUNK), jnp.float32),
            pltpu.VMEM((CHUNK, D_FEAT), jnp.float32),
            pltpu.VMEM((CHUNK, D_FEAT), jnp.float32),
            pltpu.VMEM((D_FEAT,), jnp.float32),
            pltpu.SemaphoreType.DMA,
            pltpu.SemaphoreType.DMA,
        ],
        compiler_params=pltpu.CompilerParams(needs_layout_passes=False),
    )
    return grid_kernel(z1, z2f, tau16, e0, e1, g0, g1)


def kernel(z1, z2, temp, edge_index):
    n_edges = edge_index.shape[1]
    tau = jnp.asarray(temp, dtype=jnp.float32)
    tau16 = jnp.full((L,), tau, dtype=jnp.float32)

    u = jax.random.uniform(jax.random.key(42), (n_edges, 2),
                           minval=1e-10, maxval=1.0)
    g = -jnp.log(-jnp.log(u))

    pad = E_PAD - n_edges
    e0 = jnp.pad(edge_index[0], (0, pad)).reshape(E_PAD // CHUNK, CHUNK)
    e1 = jnp.pad(edge_index[1], (0, pad)).reshape(E_PAD // CHUNK, CHUNK)
    g0 = jnp.pad(g[:, 0], (0, pad)).reshape(E_PAD // CHUNK, CHUNK)
    g1 = jnp.pad(g[:, 1], (0, pad)).reshape(E_PAD // CHUNK, CHUNK)
    z2f = z2.reshape(-1)

    out = _decode(z1, z2f, tau16, e0, e1, g0, g1)
    return out.reshape(-1)[:n_edges]


# speedup vs baseline: 4.8814x; 1.2355x over previous
import functools

import jax
import jax.numpy as jnp
from jax import lax
from jax.experimental import pallas as pl
from jax.experimental.pallas import tpu as pltpu
from jax.experimental.pallas import tpu_sc as plsc

N_NODES = 10000
D_FEAT = 256
L = 16
NW = 32
CHUNK = 64
K_CHUNKS = 80
E_PAD = NW * K_CHUNKS * CHUNK
NB = CHUNK // L


def _edge_body(z1_hbm, z2_hbm, tau_hbm, e0_hbm, e1_hbm, g0_hbm, g1_hbm,
               out_hbm, z2_v, tau_v, e0_v, e1_v, g0_v, g1_v, out_v,
               r0_v, r1_v, col_v, sem0, sem1):
    wid = lax.axis_index("s") * 2 + lax.axis_index("c")
    base = wid * K_CHUNKS

    pltpu.sync_copy(z2_hbm, z2_v)
    pltpu.sync_copy(tau_hbm, tau_v)
    pltpu.sync_copy(e0_hbm.at[pl.ds(base, K_CHUNKS)], e0_v)
    pltpu.sync_copy(e1_hbm.at[pl.ds(base, K_CHUNKS)], e1_v)
    pltpu.sync_copy(g0_hbm.at[pl.ds(base, K_CHUNKS)], g0_v)
    pltpu.sync_copy(g1_hbm.at[pl.ds(base, K_CHUNKS)], g1_v)
    tau = tau_v[...]
    lane = lax.iota(jnp.int32, L)

    pltpu.async_copy(z1_hbm.at[e0_v.at[0]], r0_v.at[0], sem0.at[0])
    pltpu.async_copy(z1_hbm.at[e1_v.at[0]], r1_v.at[0], sem1.at[0])

    def chunk_body(c, carry):
        cur = lax.rem(c, 2)
        nxt = 1 - cur

        @pl.when(c + 1 < K_CHUNKS)
        def _prefetch():
            pltpu.async_copy(z1_hbm.at[e0_v.at[c + 1]], r0_v.at[nxt],
                             sem0.at[nxt])
            pltpu.async_copy(z1_hbm.at[e1_v.at[c + 1]], r1_v.at[nxt],
                             sem1.at[nxt])

        pltpu.make_async_copy(z1_hbm.at[e0_v.at[c]], r0_v.at[cur],
                              sem0.at[cur]).wait()
        pltpu.make_async_copy(z1_hbm.at[e1_v.at[c]], r1_v.at[cur],
                              sem1.at[cur]).wait()
        for b in range(NB):
            def edge_body(i, _):
                e = b * L + i
                acc = r0_v[cur, e, pl.ds(0, L)] * r1_v[cur, e, pl.ds(0, L)]
                for j in range(1, D_FEAT // L):
                    acc = acc + (r0_v[cur, e, pl.ds(j * L, L)] *
                                 r1_v[cur, e, pl.ds(j * L, L)])
                plsc.store_scatter(col_v, [lane * L + i], acc)
                return 0

            lax.fori_loop(0, L, edge_body, 0, unroll=False)
            dot = col_v[pl.ds(0, L)]
            for d2 in range(1, L):
                dot = dot + col_v[pl.ds(d2 * L, L)]
            e0b = e0_v[c, pl.ds(b * L, L)]
            e1b = e1_v[c, pl.ds(b * L, L)]
            vn = (plsc.load_gather(z2_v, [e0b]) +
                  plsc.load_gather(z2_v, [e1b]))
            g0b = g0_v[c, pl.ds(b * L, L)]
            g1b = g1_v[c, pl.ds(b * L, L)]
            flag = (dot + g0b) / tau >= g1b / tau
            sig_f = 1.0 / (1.0 + jnp.exp(-dot))
            sig_n = 1.0 / (1.0 + jnp.exp(-vn))
            out_v[c, pl.ds(b * L, L)] = jnp.where(flag, sig_f, sig_n)
        return 0

    lax.fori_loop(0, K_CHUNKS, chunk_body, 0, unroll=False)
    pltpu.sync_copy(out_v, out_hbm.at[pl.ds(base, K_CHUNKS)])


@functools.partial(jax.jit, static_argnames=())
def _decode(z1, z2f, tau16, e0, e1, g0, g1):
    mesh = plsc.VectorSubcoreMesh(core_axis_name="c", subcore_axis_name="s")
    rows2 = E_PAD // CHUNK
    grid_kernel = pl.kernel(
        _edge_body,
        out_type=jax.ShapeDtypeStruct((rows2, CHUNK), jnp.float32),
        mesh=mesh,
        scratch_types=[
            pltpu.VMEM((N_NODES,), jnp.float32),
            pltpu.VMEM((L,), jnp.float32),
            pltpu.VMEM((K_CHUNKS, CHUNK), jnp.int32),
            pltpu.VMEM((K_CHUNKS, CHUNK), jnp.int32),
            pltpu.VMEM((K_CHUNKS, CHUNK), jnp.float32),
            pltpu.VMEM((K_CHUNKS, CHUNK), jnp.float32),
            pltpu.VMEM((K_CHUNKS, CHUNK), jnp.float32),
            pltpu.VMEM((2, CHUNK, D_FEAT), jnp.float32),
            pltpu.VMEM((2, CHUNK, D_FEAT), jnp.float32),
            pltpu.VMEM((D_FEAT,), jnp.float32),
            pltpu.SemaphoreType.DMA((2,)),
            pltpu.SemaphoreType.DMA((2,)),
        ],
        compiler_params=pltpu.CompilerParams(needs_layout_passes=False),
    )
    return grid_kernel(z1, z2f, tau16, e0, e1, g0, g1)


def kernel(z1, z2, temp, edge_index):
    n_edges = edge_index.shape[1]
    tau = jnp.asarray(temp, dtype=jnp.float32)
    tau16 = jnp.full((L,), tau, dtype=jnp.float32)

    u = jax.random.uniform(jax.random.key(42), (n_edges, 2),
                           minval=1e-10, maxval=1.0)
    g = -jnp.log(-jnp.log(u))

    pad = E_PAD - n_edges
    e0 = jnp.pad(edge_index[0], (0, pad)).reshape(E_PAD // CHUNK, CHUNK)
    e1 = jnp.pad(edge_index[1], (0, pad)).reshape(E_PAD // CHUNK, CHUNK)
    g0 = jnp.pad(g[:, 0], (0, pad)).reshape(E_PAD // CHUNK, CHUNK)
    g1 = jnp.pad(g[:, 1], (0, pad)).reshape(E_PAD // CHUNK, CHUNK)
    z2f = z2.reshape(-1)

    out = _decode(z1, z2f, tau16, e0, e1, g0, g1)
    return out.reshape(-1)[:n_edges]
